# baseline (speedup 1.0000x reference)
import functools

import numpy as np

import jax
import jax.numpy as jnp
from jax import lax
from jax.experimental import pallas as pl
from jax.experimental.pallas import tpu as pltpu
from jax.experimental.pallas import tpu_sc as plsc

_N = 50000
_D = 256
_H = 128
_NC = 16
_LANES = 128
_ROWS = 392
_NPAD = _ROWS * _LANES
_TILE = 2000
_GRID = _N // _TILE
_IMAX = 2147483647
_HI = 0x3F800002
_TBL = np.array([int(round(5000 * (s / _N))) for s in range(_N + 1)],
                dtype=np.int32)


def _score_kernel(x_ref, w1_ref, b1_ref, w2t_ref, out_ref):
    h = lax.dot_general(x_ref[...], w1_ref[...], (((1,), (1,)), ((), ())),
                        preferred_element_type=jnp.float32)
    h = h + b1_ref[...]
    out_ref[...] = lax.dot_general(h, w2t_ref[...], (((1,), (0,)), ((), ())),
                                   preferred_element_type=jnp.float32)


def _select_kernel(k_ref, b2_ref, s_ref, c_ref, tbl_ref, out_ref):
    scores = jax.nn.sigmoid(s_ref[...] + b2_ref[0])
    bits = lax.bitcast_convert_type(scores, jnp.int32)
    c = c_ref[...]
    row = lax.broadcasted_iota(jnp.int32, (_ROWS, _LANES), 0)
    col = lax.broadcasted_iota(jnp.int32, (_ROWS, _LANES), 1)
    idx = row * _LANES + col
    k = k_ref[0, 0]

    tbl = tbl_ref[...]
    sizes = [jnp.sum((c == ci).astype(jnp.int32)) for ci in range(_NC)]
    nraw = [jnp.sum(jnp.where(idx == s, tbl, 0)) for s in sizes]
    nsel = []
    pref = jnp.int32(0)
    for ci in range(_NC):
        rem = jnp.maximum(k - pref, 0)
        nsel.append(jnp.minimum(nraw[ci], rem))
        pref = pref + nraw[ci]

    def vbody(_, st):
        los, his = st[:_NC], st[_NC:]
        nlo, nhi = [], []
        for ci in range(_NC):
            mid = los[ci] + (his[ci] - los[ci]) // 2
            cnt = jnp.sum(((bits > mid) & (c == ci)).astype(jnp.int32))
            ge = cnt >= nsel[ci]
            nlo.append(jnp.where(ge, mid, los[ci]))
            nhi.append(jnp.where(ge, his[ci], mid))
        return tuple(nlo) + tuple(nhi)

    st = tuple([jnp.int32(-1)] * _NC) + tuple([jnp.int32(_HI)] * _NC)
    st = lax.fori_loop(0, 31, vbody, st)
    vth = [st[_NC + ci] for ci in range(_NC)]
    agt = [jnp.sum(((bits > vth[ci]) & (c == ci)).astype(jnp.int32))
           for ci in range(_NC)]
    mtie = [nsel[ci] - agt[ci] for ci in range(_NC)]

    def ibody(_, st):
        los, his = st[:_NC], st[_NC:]
        nlo, nhi = [], []
        for ci in range(_NC):
            mid = (los[ci] + his[ci]) // 2
            cnt = jnp.sum(((bits == vth[ci]) & (c == ci) & (idx < mid))
                          .astype(jnp.int32))
            ge = cnt >= mtie[ci]
            nlo.append(jnp.where(ge, los[ci], mid))
            nhi.append(jnp.where(ge, mid, his[ci]))
        return tuple(nlo) + tuple(nhi)

    st = tuple([jnp.int32(0)] * _NC) + tuple([jnp.int32(131072)] * _NC)
    st = lax.fori_loop(0, 17, ibody, st)
    icut = [st[_NC + ci] for ci in range(_NC)]

    sel = jnp.zeros((_ROWS, _LANES), dtype=jnp.bool_)
    for ci in range(_NC):
        use = nsel[ci] >= 1
        v = jnp.where(use, vth[ci], _IMAX)
        i = jnp.where(use, icut[ci], 0)
        sel = sel | ((c == ci) & ((bits > v) | ((bits == v) & (idx < i))))

    remaining = k - sum(nsel)
    unsel = (c >= 0) & jnp.logical_not(sel)

    def v2body(_, st):
        lo, hi = st
        mid = lo + (hi - lo) // 2
        cnt = jnp.sum(((bits > mid) & unsel).astype(jnp.int32))
        ge = cnt >= remaining
        return (jnp.where(ge, mid, lo), jnp.where(ge, hi, mid))

    lo2, hi2 = lax.fori_loop(0, 31, v2body, (jnp.int32(-1), jnp.int32(_HI)))
    agt2 = jnp.sum(((bits > hi2) & unsel).astype(jnp.int32))
    m2 = remaining - agt2

    def i2body(_, st):
        lo, hi = st
        mid = (lo + hi) // 2
        cnt = jnp.sum(((bits == hi2) & unsel & (idx < mid)).astype(jnp.int32))
        ge = cnt >= m2
        return (jnp.where(ge, lo, mid), jnp.where(ge, mid, hi))

    _, i2 = lax.fori_loop(0, 17, i2body, (jnp.int32(0), jnp.int32(131072)))

    use2 = remaining >= 1
    v2 = jnp.where(use2, hi2, _IMAX)
    i2 = jnp.where(use2, i2, 0)
    sel = sel | (unsel & ((bits > v2) | ((bits == v2) & (idx < i2))))
    out_ref[...] = sel.astype(jnp.float32)


def kernel(x, c, k, W1, b1, W2, b2):
    logits = pl.pallas_call(
        _score_kernel,
        grid=(_GRID,),
        in_specs=[
            pl.BlockSpec((_TILE, _D), lambda i: (i, 0)),
            pl.BlockSpec((_H, _D), lambda i: (0, 0)),
            pl.BlockSpec((1, _H), lambda i: (0, 0)),
            pl.BlockSpec((_H, 1), lambda i: (0, 0)),
        ],
        out_specs=pl.BlockSpec((_TILE, 1), lambda i: (i, 0)),
        out_shape=jax.ShapeDtypeStruct((_N, 1), jnp.float32),
    )(x, W1, b1.reshape(1, _H), W2.reshape(_H, 1))

    s_p = jnp.pad(logits[:, 0], (0, _NPAD - _N),
                  constant_values=0.0).reshape(_ROWS, _LANES)
    c_p = jnp.pad(c, (0, _NPAD - _N), constant_values=-1).reshape(_ROWS, _LANES)
    k_arr = jnp.asarray(k, jnp.int32).reshape(1, 1)
    b2_arr = jnp.asarray(b2, jnp.float32).reshape(1)
    tbl_p = jnp.asarray(
        np.pad(_TBL, (0, _NPAD - _N - 1)).reshape(_ROWS, _LANES))

    sel = pl.pallas_call(
        _select_kernel,
        in_specs=[
            pl.BlockSpec(memory_space=pltpu.SMEM),
            pl.BlockSpec(memory_space=pltpu.SMEM),
            pl.BlockSpec((_ROWS, _LANES), lambda: (0, 0)),
            pl.BlockSpec((_ROWS, _LANES), lambda: (0, 0)),
            pl.BlockSpec((_ROWS, _LANES), lambda: (0, 0)),
        ],
        out_specs=pl.BlockSpec((_ROWS, _LANES), lambda: (0, 0)),
        out_shape=jax.ShapeDtypeStruct((_ROWS, _LANES), jnp.float32),
    )(k_arr, b2_arr, s_p, c_p, tbl_p)
    return sel.reshape(-1)[:_N, None]


# speedup vs baseline: 1.1520x; 1.1520x over previous
import functools

import numpy as np

import jax
import jax.numpy as jnp
from jax import lax
from jax.experimental import pallas as pl
from jax.experimental.pallas import tpu as pltpu
from jax.experimental.pallas import tpu_sc as plsc

_N = 50000
_D = 256
_H = 128
_NC = 16
_LANES = 128
_ROWS = 392
_NPAD = _ROWS * _LANES
_TILE = 2000
_GRID = _N // _TILE
_IMAX = 2147483647
_HI = 0x3F800002
_TBL = np.array([int(round(5000 * (s / _N))) for s in range(_N + 1)],
                dtype=np.int32)


def _score_kernel(x_ref, w1_ref, b1_ref, w2t_ref, out_ref):
    h = lax.dot_general(x_ref[...], w1_ref[...], (((1,), (1,)), ((), ())),
                        preferred_element_type=jnp.float32)
    h = h + b1_ref[...]
    out_ref[...] = lax.dot_general(h, w2t_ref[...], (((1,), (0,)), ((), ())),
                                   preferred_element_type=jnp.float32)


def _select_kernel(k_ref, b2_ref, s_ref, c_ref, tbl_ref, out_ref):
    scores = jax.nn.sigmoid(s_ref[...] + b2_ref[0])
    bits = lax.bitcast_convert_type(scores, jnp.int32)
    c = c_ref[...]
    row = lax.broadcasted_iota(jnp.int32, (_ROWS, _LANES), 0)
    col = lax.broadcasted_iota(jnp.int32, (_ROWS, _LANES), 1)
    idx = row * _LANES + col
    k = k_ref[0, 0]

    tbl = tbl_ref[...]
    sizes = [jnp.sum((c == ci).astype(jnp.int32)) for ci in range(_NC)]
    nraw = [jnp.sum(jnp.where(idx == s, tbl, 0)) for s in sizes]
    nsel = []
    pref = jnp.int32(0)
    for ci in range(_NC):
        rem = jnp.maximum(k - pref, 0)
        nsel.append(jnp.minimum(nraw[ci], rem))
        pref = pref + nraw[ci]

    mb = [jnp.where(c == ci, bits, -1) for ci in range(_NC)]

    def vbody(_, st):
        los, his = st[:_NC], st[_NC:]
        nlo, nhi = [], []
        for ci in range(_NC):
            mid = los[ci] + (his[ci] - los[ci]) // 2
            cnt = jnp.sum((mb[ci] > mid).astype(jnp.int32))
            ge = cnt >= nsel[ci]
            nlo.append(jnp.where(ge, mid, los[ci]))
            nhi.append(jnp.where(ge, his[ci], mid))
        return tuple(nlo) + tuple(nhi)

    st = tuple([jnp.int32(-1)] * _NC) + tuple([jnp.int32(_HI)] * _NC)
    st = lax.fori_loop(0, 31, vbody, st)
    vth = [st[_NC + ci] for ci in range(_NC)]
    agt = [jnp.sum((mb[ci] > vth[ci]).astype(jnp.int32)) for ci in range(_NC)]
    mtie = [nsel[ci] - agt[ci] for ci in range(_NC)]

    mi = [jnp.where(mb[ci] == vth[ci], idx, _IMAX) for ci in range(_NC)]

    def ibody(_, st):
        los, his = st[:_NC], st[_NC:]
        nlo, nhi = [], []
        for ci in range(_NC):
            mid = (los[ci] + his[ci]) // 2
            cnt = jnp.sum((mi[ci] < mid).astype(jnp.int32))
            ge = cnt >= mtie[ci]
            nlo.append(jnp.where(ge, los[ci], mid))
            nhi.append(jnp.where(ge, mid, his[ci]))
        return tuple(nlo) + tuple(nhi)

    st = tuple([jnp.int32(0)] * _NC) + tuple([jnp.int32(131072)] * _NC)
    st = lax.fori_loop(0, 17, ibody, st)
    icut = [st[_NC + ci] for ci in range(_NC)]

    sel = jnp.zeros((_ROWS, _LANES), dtype=jnp.bool_)
    for ci in range(_NC):
        use = nsel[ci] >= 1
        v = jnp.where(use, vth[ci], _IMAX)
        i = jnp.where(use, icut[ci], 0)
        sel = sel | (mb[ci] > v) | ((mb[ci] == v) & (idx < i))

    remaining = k - sum(nsel)
    unsel = (c >= 0) & jnp.logical_not(sel)
    mb2 = jnp.where(unsel, bits, -1)

    def v2body(_, st):
        lo, hi = st
        mid = lo + (hi - lo) // 2
        cnt = jnp.sum((mb2 > mid).astype(jnp.int32))
        ge = cnt >= remaining
        return (jnp.where(ge, mid, lo), jnp.where(ge, hi, mid))

    lo2, hi2 = lax.fori_loop(0, 31, v2body, (jnp.int32(-1), jnp.int32(_HI)))
    agt2 = jnp.sum((mb2 > hi2).astype(jnp.int32))
    m2 = remaining - agt2
    mi2 = jnp.where(mb2 == hi2, idx, _IMAX)

    def i2body(_, st):
        lo, hi = st
        mid = (lo + hi) // 2
        cnt = jnp.sum((mi2 < mid).astype(jnp.int32))
        ge = cnt >= m2
        return (jnp.where(ge, lo, mid), jnp.where(ge, mid, hi))

    _, i2 = lax.fori_loop(0, 17, i2body, (jnp.int32(0), jnp.int32(131072)))

    use2 = remaining >= 1
    v2 = jnp.where(use2, hi2, _IMAX)
    i2 = jnp.where(use2, i2, 0)
    sel = sel | (unsel & ((bits > v2) | ((bits == v2) & (idx < i2))))
    out_ref[...] = sel.astype(jnp.float32)


def kernel(x, c, k, W1, b1, W2, b2):
    logits = pl.pallas_call(
        _score_kernel,
        grid=(_GRID,),
        in_specs=[
            pl.BlockSpec((_TILE, _D), lambda i: (i, 0)),
            pl.BlockSpec((_H, _D), lambda i: (0, 0)),
            pl.BlockSpec((1, _H), lambda i: (0, 0)),
            pl.BlockSpec((_H, 1), lambda i: (0, 0)),
        ],
        out_specs=pl.BlockSpec((_TILE, 1), lambda i: (i, 0)),
        out_shape=jax.ShapeDtypeStruct((_N, 1), jnp.float32),
    )(x, W1, b1.reshape(1, _H), W2.reshape(_H, 1))

    s_p = jnp.pad(logits[:, 0], (0, _NPAD - _N),
                  constant_values=0.0).reshape(_ROWS, _LANES)
    c_p = jnp.pad(c, (0, _NPAD - _N), constant_values=-1).reshape(_ROWS, _LANES)
    k_arr = jnp.asarray(k, jnp.int32).reshape(1, 1)
    b2_arr = jnp.asarray(b2, jnp.float32).reshape(1)
    tbl_p = jnp.asarray(
        np.pad(_TBL, (0, _NPAD - _N - 1)).reshape(_ROWS, _LANES))

    sel = pl.pallas_call(
        _select_kernel,
        in_specs=[
            pl.BlockSpec(memory_space=pltpu.SMEM),
            pl.BlockSpec(memory_space=pltpu.SMEM),
            pl.BlockSpec((_ROWS, _LANES), lambda: (0, 0)),
            pl.BlockSpec((_ROWS, _LANES), lambda: (0, 0)),
            pl.BlockSpec((_ROWS, _LANES), lambda: (0, 0)),
        ],
        out_specs=pl.BlockSpec((_ROWS, _LANES), lambda: (0, 0)),
        out_shape=jax.ShapeDtypeStruct((_ROWS, _LANES), jnp.float32),
    )(k_arr, b2_arr, s_p, c_p, tbl_p)
    return sel.reshape(-1)[:_N, None]
